# Initial kernel scaffold; baseline (speedup 1.0000x reference)
import jax
import jax.numpy as jnp
from jax.experimental import pallas as pl


def kernel(logits, target, mask):
    raise NotImplementedError("write your pallas kernel here")


# speedup vs baseline: 1.1571x; 1.1571x over previous
import jax
import jax.numpy as jnp
from jax import lax
from jax.experimental import pallas as pl
from jax.experimental.pallas import tpu as pltpu
from jax.experimental.pallas import tpu_sc as plsc

_ACC_K = 5
_V = 100000
_CHUNK = 20000
_NCHUNK = _V // _CHUNK
_VECS = _CHUNK // 16
_NC = 2
_NS = 16
_NW = _NC * _NS
_N = 256
_RPW = _N // _NW


def _body(logits_hbm, tgt_hbm, mask_hbm, out_hbm,
          buf0, buf1, win_v, tgt_v, mask_v, res_v, sem0, sem1):
    cid = lax.axis_index("c")
    sid = lax.axis_index("s")
    wid = sid * _NC + cid
    base = wid * _RPW

    pltpu.sync_copy(tgt_hbm.at[pl.ds(pl.multiple_of(base, 8), _RPW)],
                    tgt_v.at[pl.ds(0, _RPW)])
    pltpu.sync_copy(mask_hbm.at[pl.ds(pl.multiple_of(base, 8), _RPW)],
                    mask_v.at[pl.ds(0, _RPW)])
    tgt_all = tgt_v[...]
    mask_all = mask_v[...]

    for j in range(_RPW):
        t = tgt_all[j]
        ta = t & -8
        start = pl.multiple_of((base + j) * _V + ta, 8)
        pltpu.sync_copy(logits_hbm.at[pl.ds(start, 8)],
                        win_v.at[pl.ds(j * 8, 8)])

    lane = lax.iota(jnp.int32, 16)
    zeros_i = jnp.zeros((16,), jnp.int32)
    ones_i = jnp.ones((16,), jnp.int32)
    zeros_f = jnp.zeros((16,), jnp.float32)

    bufs = (buf0, buf1)
    sems = (sem0, sem1)
    total = _RPW * _NCHUNK

    def chunk_src(k):
        j, c = divmod(k, _NCHUNK)
        off = pl.multiple_of((base + j) * _V + c * _CHUNK, 8)
        return logits_hbm.at[pl.ds(off, _CHUNK)]

    cp = pltpu.async_copy(chunk_src(0), bufs[0], sems[0])

    res = zeros_f
    for j in range(_RPW):
        t = tgt_all[j]
        tvec = jnp.full((16,), t, jnp.int32)
        vt = plsc.load_gather(
            win_v, [jnp.full((16,), j * 8, jnp.int32) + (t & 7)])
        count = zeros_i
        for c in range(_NCHUNK):
            k = j * _NCHUNK + c
            if k + 1 < total:
                nxt = pltpu.async_copy(
                    chunk_src(k + 1), bufs[(k + 1) % 2], sems[(k + 1) % 2])
            cp.wait()
            buf = bufs[k % 2]
            cbase = c * _CHUNK

            def tick(i, cnt):
                v = buf[pl.ds(pl.multiple_of(i * 16, 16), 16)]
                pos = lane + (cbase + i * 16)
                gt = v > vt
                eq = v == vt
                sel = pos < tvec
                take = gt | (eq & sel)
                return cnt + jnp.where(take, ones_i, zeros_i)

            count = lax.fori_loop(0, _VECS, tick, count)
            if k + 1 < total:
                cp = nxt
        rank = jnp.sum(count)
        mf = mask_all[j].astype(jnp.float32)
        hitm = jnp.where(rank < _ACC_K, mf, jnp.float32(0.0))
        res = res + jnp.where(lane == j, jnp.full((16,), hitm), zeros_f)
        res = res + jnp.where(lane == (8 + j), jnp.full((16,), mf), zeros_f)

    res_v[...] = res
    pltpu.sync_copy(res_v, out_hbm.at[wid])


_sc_call = pl.kernel(
    _body,
    out_type=jax.ShapeDtypeStruct((_NW, 16), jnp.float32),
    mesh=plsc.VectorSubcoreMesh(core_axis_name="c", subcore_axis_name="s"),
    compiler_params=pltpu.CompilerParams(needs_layout_passes=False),
    scratch_types=[
        pltpu.VMEM((_CHUNK,), jnp.float32),
        pltpu.VMEM((_CHUNK,), jnp.float32),
        pltpu.VMEM((_RPW * 8,), jnp.float32),
        pltpu.VMEM((16,), jnp.int32),
        pltpu.VMEM((16,), jnp.int32),
        pltpu.VMEM((16,), jnp.float32),
        pltpu.SemaphoreType.DMA,
        pltpu.SemaphoreType.DMA,
    ],
)


def kernel(logits, target, mask):
    flat_logits = logits.reshape(-1)
    tgt = target.reshape(-1).astype(jnp.int32)
    msk = mask.reshape(-1).astype(jnp.int32)
    part = _sc_call(flat_logits, tgt, msk)
    counter = jnp.sum(part[:, :_RPW])
    all_counter = jnp.sum(part[:, _RPW:])
    return (counter / all_counter)[None].astype(jnp.float32)


# speedup vs baseline: 1.7858x; 1.5433x over previous
import jax
import jax.numpy as jnp
from jax import lax
from jax.experimental import pallas as pl
from jax.experimental.pallas import tpu as pltpu
from jax.experimental.pallas import tpu_sc as plsc

_ACC_K = 5
_V = 100000
_CHUNK = 20000
_NCHUNK = _V // _CHUNK
_VECS = _CHUNK // 16
_NC = 2
_NS = 16
_NW = _NC * _NS
_N = 256
_RPW = _N // _NW


def _body(logits_hbm, tgt_hbm, mask_hbm, out_hbm,
          buf0, buf1, win_v, tgt_v, mask_v, res_v, sem0, sem1):
    cid = lax.axis_index("c")
    sid = lax.axis_index("s")
    wid = sid * _NC + cid
    base = wid * _RPW

    pltpu.sync_copy(tgt_hbm.at[pl.ds(pl.multiple_of(base, 8), _RPW)],
                    tgt_v.at[pl.ds(0, _RPW)])
    pltpu.sync_copy(mask_hbm.at[pl.ds(pl.multiple_of(base, 8), _RPW)],
                    mask_v.at[pl.ds(0, _RPW)])
    tgt_all = tgt_v[...]
    mask_all = mask_v[...]

    for j in range(_RPW):
        t = tgt_all[j]
        ta = t & -8
        start = pl.multiple_of((base + j) * _V + ta, 8)
        pltpu.sync_copy(logits_hbm.at[pl.ds(start, 8)],
                        win_v.at[pl.ds(j * 8, 8)])

    lane = lax.iota(jnp.int32, 16)
    zeros_i = jnp.zeros((16,), jnp.int32)
    ones_i = jnp.ones((16,), jnp.int32)
    zeros_f = jnp.zeros((16,), jnp.float32)

    bufs = (buf0, buf1)
    sems = (sem0, sem1)
    total = _RPW * _NCHUNK

    def chunk_src(k):
        j, c = divmod(k, _NCHUNK)
        off = pl.multiple_of((base + j) * _V + c * _CHUNK, 8)
        return logits_hbm.at[pl.ds(off, _CHUNK)]

    cp = pltpu.async_copy(chunk_src(0), bufs[0], sems[0])

    res = zeros_f
    for j in range(_RPW):
        t = tgt_all[j]
        tvec = jnp.full((16,), t, jnp.int32)
        vt = plsc.load_gather(
            win_v, [jnp.full((16,), j * 8, jnp.int32) + (t & 7)])
        count = zeros_i
        for c in range(_NCHUNK):
            k = j * _NCHUNK + c
            if k + 1 < total:
                nxt = pltpu.async_copy(
                    chunk_src(k + 1), bufs[(k + 1) % 2], sems[(k + 1) % 2])
            cp.wait()
            buf = bufs[k % 2]
            cbase = c * _CHUNK

            t_rel = jnp.clip(t - cbase, 0, _CHUNK)
            nge = lax.shift_right_logical(t_rel, 4)

            @plsc.parallel_loop(0, nge, 1, unroll=8, carry=count)
            def ge_loop(i, cnt):
                v = buf[pl.ds(pl.multiple_of(i * 16, 16), 16)]
                return cnt + jnp.where(v >= vt, ones_i, zeros_i)

            count = ge_loop
            bidx = jnp.minimum(nge, _VECS - 1)
            v = buf[pl.ds(pl.multiple_of(bidx * 16, 16), 16)]
            pos = lane + (cbase + bidx * 16)
            mixed = (v > vt) | ((v == vt) & (pos < tvec))
            bcnt = jnp.where(mixed, ones_i, zeros_i)
            guard = jnp.full((16,), nge < _VECS)
            count = count + jnp.where(guard, bcnt, zeros_i)

            @plsc.parallel_loop(nge + 1, _VECS, 1, unroll=8, carry=count)
            def gt_loop(i, cnt):
                v = buf[pl.ds(pl.multiple_of(i * 16, 16), 16)]
                return cnt + jnp.where(v > vt, ones_i, zeros_i)

            count = gt_loop
            if k + 1 < total:
                cp = nxt
        rank = jnp.sum(count)
        mf = mask_all[j].astype(jnp.float32)
        hitm = jnp.where(rank < _ACC_K, mf, jnp.float32(0.0))
        res = res + jnp.where(lane == j, jnp.full((16,), hitm), zeros_f)
        res = res + jnp.where(lane == (8 + j), jnp.full((16,), mf), zeros_f)

    res_v[...] = res
    pltpu.sync_copy(res_v, out_hbm.at[wid])


_sc_call = pl.kernel(
    _body,
    out_type=jax.ShapeDtypeStruct((_NW, 16), jnp.float32),
    mesh=plsc.VectorSubcoreMesh(core_axis_name="c", subcore_axis_name="s"),
    compiler_params=pltpu.CompilerParams(needs_layout_passes=False),
    scratch_types=[
        pltpu.VMEM((_CHUNK,), jnp.float32),
        pltpu.VMEM((_CHUNK,), jnp.float32),
        pltpu.VMEM((_RPW * 8,), jnp.float32),
        pltpu.VMEM((16,), jnp.int32),
        pltpu.VMEM((16,), jnp.int32),
        pltpu.VMEM((16,), jnp.float32),
        pltpu.SemaphoreType.DMA,
        pltpu.SemaphoreType.DMA,
    ],
)


def kernel(logits, target, mask):
    flat_logits = logits.reshape(-1)
    tgt = target.reshape(-1).astype(jnp.int32)
    msk = mask.reshape(-1).astype(jnp.int32)
    part = _sc_call(flat_logits, tgt, msk)
    counter = jnp.sum(part[:, :_RPW])
    all_counter = jnp.sum(part[:, _RPW:])
    return (counter / all_counter)[None].astype(jnp.float32)
